# baseline (speedup 1.0000x reference)
import functools

import jax
import jax.numpy as jnp
from jax import lax
from jax.experimental import pallas as pl
from jax.experimental.pallas import tpu as pltpu


def _index_kernel(hist_ref, idx_ref):
    B, L = hist_ref.shape
    hist = hist_ref[...]
    pos = lax.broadcasted_iota(jnp.int32, (B, L), 1)
    cand = jnp.where(hist > 0, pos, -1)
    idx = jnp.max(cand, axis=1)
    idx_ref[...] = jnp.maximum(idx, 0)


def _broadcast_kernel(idx_ref, vals_ref, out_ref):
    b = pl.program_id(0)
    r = idx_ref[b] % 8
    row = vals_ref[0, pl.ds(r, 1), :]
    out_ref[...] = jnp.broadcast_to(row[None], out_ref.shape)


def kernel(input_values, input_timestamps, is_target_mask, dummy):
    B, L, D = input_values.shape
    hist = jnp.logical_not(is_target_mask).astype(jnp.int32)

    idx = pl.pallas_call(
        _index_kernel,
        out_shape=jax.ShapeDtypeStruct((B,), jnp.int32),
    )(hist)

    grid_spec = pltpu.PrefetchScalarGridSpec(
        num_scalar_prefetch=1,
        grid=(B,),
        in_specs=[
            pl.BlockSpec((1, 8, D), lambda b, idx_ref: (b, idx_ref[b] // 8, 0)),
        ],
        out_specs=pl.BlockSpec((1, L, D), lambda b, idx_ref: (b, 0, 0)),
    )
    out = pl.pallas_call(
        _broadcast_kernel,
        grid_spec=grid_spec,
        out_shape=jax.ShapeDtypeStruct((B, L, D), jnp.float32),
    )(idx, input_values)
    return out
